# Initial kernel scaffold; baseline (speedup 1.0000x reference)
import jax
import jax.numpy as jnp
from jax.experimental import pallas as pl


def kernel(xyz, points, params):
    raise NotImplementedError("write your pallas kernel here")


# speedup vs baseline: 17.2040x; 17.2040x over previous
import functools

import jax
import jax.numpy as jnp
import numpy as np
from jax import lax
from jax.experimental import pallas as pl
from jax.experimental.pallas import tpu as pltpu
from jax.experimental.pallas import tpu_sc as plsc

B = 8
N = 4096
S = 512
KS = (16, 32, 64)
R2 = tuple(np.float32(r * r) for r in (0.1, 0.2, 0.4))
KTOT = sum(KS)
GW = 3 * KTOT
XOFF = (0, KS[0], KS[0] + KS[1])


def _fps_body(xyz_ref, new_ref, idx_ref):
    x = xyz_ref[:, 0, :]
    y = xyz_ref[:, 1, :]
    z = xyz_ref[:, 2, :]
    iota = lax.broadcasted_iota(jnp.int32, (B, N), 1)
    slot = (lax.broadcasted_iota(jnp.int32, (B, S), 1)
            + lax.broadcasted_iota(jnp.int32, (B, S), 0) * S) & (S - 1)

    def step(t, carry):
        dist, idx, cx, cy, cz, ia, xa, ya, za = carry
        here = slot == t
        hi = here.astype(jnp.int32)
        hf = here.astype(jnp.float32)
        ia = ia + hi * idx
        xa = xa + hf * cx
        ya = ya + hf * cy
        za = za + hf * cz
        d = (x - cx) ** 2 + (y - cy) ** 2 + (z - cz) ** 2
        dist = jnp.minimum(dist, d)
        m = jnp.max(dist, axis=1, keepdims=True)
        cand = jnp.where(dist == m, iota, N)
        nidx = jnp.min(cand, axis=1, keepdims=True)
        sel = iota == nidx
        zf = jnp.zeros((), jnp.float32)
        ncx = jnp.sum(jnp.where(sel, x, zf), axis=1, keepdims=True)
        ncy = jnp.sum(jnp.where(sel, y, zf), axis=1, keepdims=True)
        ncz = jnp.sum(jnp.where(sel, z, zf), axis=1, keepdims=True)
        return dist, nidx, ncx, ncy, ncz, ia, xa, ya, za

    init = (
        jnp.full((B, N), 1e10, jnp.float32),
        jnp.zeros((B, 1), jnp.int32),
        x[:, 0:1],
        y[:, 0:1],
        z[:, 0:1],
        jnp.zeros((B, S), jnp.int32),
        jnp.zeros((B, S), jnp.float32),
        jnp.zeros((B, S), jnp.float32),
        jnp.zeros((B, S), jnp.float32),
    )
    res = lax.fori_loop(0, S, step, init)
    idx_ref[...] = res[5]
    new_ref[:, 0, :] = res[6]
    new_ref[:, 1, :] = res[7]
    new_ref[:, 2, :] = res[8]


def _fps(xyz):
    return pl.pallas_call(
        _fps_body,
        out_shape=[
            jax.ShapeDtypeStruct((B, 3, S), jnp.float32),
            jax.ShapeDtypeStruct((B, S), jnp.int32),
        ],
    )(xyz)


def _make_select():
    mesh = plsc.VectorSubcoreMesh(core_axis_name="c", subcore_axis_name="s")

    @functools.partial(
        pl.kernel,
        mesh=mesh,
        compiler_params=pltpu.CompilerParams(needs_layout_passes=False),
        out_type=jax.ShapeDtypeStruct((B * S, GW), jnp.float32),
        scratch_types=[
            pltpu.VMEM((N,), jnp.float32),
            pltpu.VMEM((N,), jnp.float32),
            pltpu.VMEM((N,), jnp.float32),
            pltpu.VMEM((S,), jnp.int32),
            pltpu.VMEM((GW,), jnp.float32),
            pltpu.VMEM((KS[0],), jnp.int32),
            pltpu.VMEM((KS[1],), jnp.int32),
            pltpu.VMEM((KS[2],), jnp.int32),
        ],
    )
    def select(xyz_hbm, fps_hbm, g_hbm, xv, yv, zv, fpsv, outb, ib1, ib2, ib3):
        wid = lax.axis_index("s") * 2 + lax.axis_index("c")
        b = wid // 4
        q = wid % 4
        pltpu.sync_copy(xyz_hbm.at[b * 3 + 0], xv)
        pltpu.sync_copy(xyz_hbm.at[b * 3 + 1], yv)
        pltpu.sync_copy(xyz_hbm.at[b * 3 + 2], zv)
        pltpu.sync_copy(fps_hbm.at[b], fpsv)
        lane = jnp.arange(16, dtype=jnp.int32)
        ibufs = (ib1, ib2, ib3)

        def per_center(s_loc, _):
            s_abs = q * 128 + s_loc
            cidx = plsc.load_gather(fpsv, [jnp.full((16,), s_abs, jnp.int32)])
            cx = plsc.load_gather(xv, [cidx])
            cy = plsc.load_gather(yv, [cidx])
            cz = plsc.load_gather(zv, [cidx])
            fill = jnp.full((16,), N - 1, jnp.int32)
            for i in range(3):
                for j in range(KS[i] // 16):
                    ibufs[i][pl.ds(16 * j, 16)] = fill

            def scan_chunk(c, carry):
                cnts, firsts = carry
                xc = xv[pl.ds(c * 16, 16)]
                yc = yv[pl.ds(c * 16, 16)]
                zc = zv[pl.ds(c * 16, 16)]
                dx = xc - cx
                dy = yc - cy
                dz = zc - cz
                d2 = dx * dx + dy * dy + dz * dz
                gidx = lane + c * 16
                ncnt = []
                nfirst = []
                for i in range(3):
                    m = d2 <= R2[i]
                    cs = jnp.cumsum(m.astype(jnp.int32))
                    slot = cnts[i] + cs - 1
                    valid = jnp.logical_and(m, slot < KS[i])
                    slot = jnp.minimum(jnp.maximum(slot, 0), KS[i] - 1)
                    plsc.store_scatter(ibufs[i], [slot], gidx, mask=valid)
                    ncnt.append(cnts[i] + plsc.all_reduce_population_count(m))
                    nfirst.append(
                        jnp.minimum(firsts[i], jnp.where(m, gidx, N - 1)))
                return tuple(ncnt), tuple(nfirst)

            z16 = jnp.zeros((16,), jnp.int32)
            f16 = jnp.full((16,), N - 1, jnp.int32)
            cnts, firsts = lax.fori_loop(
                0, N // 16, scan_chunk, ((z16, z16, z16), (f16, f16, f16)))
            for i in range(3):
                first = jnp.full((16,), jnp.min(firsts[i]), jnp.int32)
                for j in range(KS[i] // 16):
                    cur = ibufs[i][pl.ds(16 * j, 16)]
                    res = jnp.where(lane + 16 * j < cnts[i], cur, first)
                    off = XOFF[i] + 16 * j
                    outb[pl.ds(off, 16)] = plsc.load_gather(xv, [res]) - cx
                    outb[pl.ds(KTOT + off, 16)] = plsc.load_gather(yv, [res]) - cy
                    outb[pl.ds(2 * KTOT + off, 16)] = plsc.load_gather(zv, [res]) - cz
            pltpu.sync_copy(outb, g_hbm.at[b * S + s_abs])
            return 0

        lax.fori_loop(0, 128, per_center, 0)

    return select


_select_cache = []


def _get_select():
    if not _select_cache:
        _select_cache.append(_make_select())
    return _select_cache[0]


SBLK = 64


def _mlp_body(g_ref, *refs):
    wrefs = refs[:18]
    o_refs = refs[18:]
    sb = pl.program_id(1)
    r = g_ref[0]
    for i in range(3):
        k = KS[i]
        o = XOFF[i]
        gx = r[:, o:o + k]
        gy = r[:, KTOT + o:KTOT + o + k]
        gz = r[:, 2 * KTOT + o:2 * KTOT + o + k]
        w1, b1, w2, b2, w3, b3 = wrefs[6 * i:6 * i + 6]
        h = (gx[:, :, None] * w1[0][None, None, :]
             + gy[:, :, None] * w1[1][None, None, :]
             + gz[:, :, None] * w1[2][None, None, :]
             + b1[0][None, None, :])
        h = jnp.maximum(h, 0.0)
        c1 = h.shape[-1]
        h = h.reshape(SBLK * k, c1)
        h = jnp.maximum(jnp.dot(h, w2[...], preferred_element_type=jnp.float32)
                        + b2[...], 0.0)
        h = jnp.maximum(jnp.dot(h, w3[...], preferred_element_type=jnp.float32)
                        + b3[...], 0.0)
        m = jnp.max(h.reshape(SBLK, k, 128), axis=0)
        o_ref = o_refs[i]

        @pl.when(sb == 0)
        def _():
            o_ref[0] = m

        @pl.when(sb != 0)
        def _():
            o_ref[0] = jnp.maximum(o_ref[0], m)


def _mlp(g, wlist):
    grid = (B, S // SBLK)
    in_specs = [pl.BlockSpec((1, SBLK, GW), lambda bb, sb: (bb, sb, 0))]
    for w in wlist:
        in_specs.append(
            pl.BlockSpec(w.shape, lambda bb, sb, nd=w.ndim: (0,) * nd))
    out_specs = [
        pl.BlockSpec((1, KS[i], 128), lambda bb, sb: (bb, 0, 0))
        for i in range(3)
    ]
    return pl.pallas_call(
        _mlp_body,
        grid=grid,
        in_specs=in_specs,
        out_specs=out_specs,
        out_shape=[
            jax.ShapeDtypeStruct((B, KS[i], 128), jnp.float32) for i in range(3)
        ],
    )(g, *wlist)


def kernel(xyz, points, params):
    del points
    new_xyz, fps_idx = _fps(xyz)
    g = _get_select()(xyz.reshape(B * 3, N), fps_idx)
    wlist = []
    for i in range(3):
        for (w, bvec) in params[i]:
            wlist.append(w)
            wlist.append(bvec.reshape(1, -1))
    o1, o2, o3 = _mlp(g.reshape(B, S, GW), wlist)
    return new_xyz, jnp.concatenate([o1, o2, o3], axis=1)


# speedup vs baseline: 21.6258x; 1.2570x over previous
import functools

import jax
import jax.numpy as jnp
import numpy as np
from jax import lax
from jax.experimental import pallas as pl
from jax.experimental.pallas import tpu as pltpu
from jax.experimental.pallas import tpu_sc as plsc

B = 8
N = 4096
S = 512
KS = (16, 32, 64)
R2 = tuple(np.float32(r * r) for r in (0.1, 0.2, 0.4))
KTOT = sum(KS)
GW = 3 * KTOT
XOFF = (0, KS[0], KS[0] + KS[1])


def _fps_body(xyz_ref, new_ref, idx_ref):
    x = xyz_ref[:, 0, :]
    y = xyz_ref[:, 1, :]
    z = xyz_ref[:, 2, :]
    iota = lax.broadcasted_iota(jnp.int32, (B, N), 1)
    slot = (lax.broadcasted_iota(jnp.int32, (B, S), 1)
            + lax.broadcasted_iota(jnp.int32, (B, S), 0) * S) & (S - 1)

    def step(t, carry):
        dist, idx, cx, cy, cz, ia, xa, ya, za = carry
        here = slot == t
        hi = here.astype(jnp.int32)
        hf = here.astype(jnp.float32)
        ia = ia + hi * idx
        xa = xa + hf * cx
        ya = ya + hf * cy
        za = za + hf * cz
        d = (x - cx) ** 2 + (y - cy) ** 2 + (z - cz) ** 2
        dist = jnp.minimum(dist, d)
        m = jnp.max(dist, axis=1, keepdims=True)
        cand = jnp.where(dist == m, iota, N)
        nidx = jnp.min(cand, axis=1, keepdims=True)
        sel = iota == nidx
        zf = jnp.zeros((), jnp.float32)
        ncx = jnp.sum(jnp.where(sel, x, zf), axis=1, keepdims=True)
        ncy = jnp.sum(jnp.where(sel, y, zf), axis=1, keepdims=True)
        ncz = jnp.sum(jnp.where(sel, z, zf), axis=1, keepdims=True)
        return dist, nidx, ncx, ncy, ncz, ia, xa, ya, za

    init = (
        jnp.full((B, N), 1e10, jnp.float32),
        jnp.zeros((B, 1), jnp.int32),
        x[:, 0:1],
        y[:, 0:1],
        z[:, 0:1],
        jnp.zeros((B, S), jnp.int32),
        jnp.zeros((B, S), jnp.float32),
        jnp.zeros((B, S), jnp.float32),
        jnp.zeros((B, S), jnp.float32),
    )
    res = lax.fori_loop(0, S, step, init)
    idx_ref[...] = res[5]
    new_ref[:, 0, :] = res[6]
    new_ref[:, 1, :] = res[7]
    new_ref[:, 2, :] = res[8]


def _fps(xyz):
    return pl.pallas_call(
        _fps_body,
        out_shape=[
            jax.ShapeDtypeStruct((B, 3, S), jnp.float32),
            jax.ShapeDtypeStruct((B, S), jnp.int32),
        ],
    )(xyz)


def _make_select():
    mesh = plsc.VectorSubcoreMesh(core_axis_name="c", subcore_axis_name="s")

    @functools.partial(
        pl.kernel,
        mesh=mesh,
        compiler_params=pltpu.CompilerParams(needs_layout_passes=False),
        out_type=jax.ShapeDtypeStruct((B * S, GW), jnp.float32),
        scratch_types=[
            pltpu.VMEM((N,), jnp.float32),
            pltpu.VMEM((N,), jnp.float32),
            pltpu.VMEM((N,), jnp.float32),
            pltpu.VMEM((S,), jnp.int32),
            pltpu.VMEM((GW,), jnp.float32),
            pltpu.VMEM((KS[0],), jnp.int32),
            pltpu.VMEM((KS[1],), jnp.int32),
            pltpu.VMEM((KS[2],), jnp.int32),
        ],
    )
    def select(xyz_hbm, fps_hbm, g_hbm, xv, yv, zv, fpsv, outb, ib1, ib2, ib3):
        wid = lax.axis_index("s") * 2 + lax.axis_index("c")
        b = wid // 4
        q = wid % 4
        pltpu.sync_copy(xyz_hbm.at[b * 3 + 0], xv)
        pltpu.sync_copy(xyz_hbm.at[b * 3 + 1], yv)
        pltpu.sync_copy(xyz_hbm.at[b * 3 + 2], zv)
        pltpu.sync_copy(fps_hbm.at[b], fpsv)
        lane = jnp.arange(16, dtype=jnp.int32)
        ibufs = (ib1, ib2, ib3)

        def per_center(s_loc, _):
            s_abs = q * 128 + s_loc
            cidx = plsc.load_gather(fpsv, [jnp.full((16,), s_abs, jnp.int32)])
            cx = plsc.load_gather(xv, [cidx])
            cy = plsc.load_gather(yv, [cidx])
            cz = plsc.load_gather(zv, [cidx])
            fill = jnp.full((16,), N - 1, jnp.int32)
            for i in range(3):
                for j in range(KS[i] // 16):
                    ibufs[i][pl.ds(16 * j, 16)] = fill

            UNROLL = 4

            def scan_group(g, carry):
                cnts, firsts = carry
                cnts = list(cnts)
                firsts = list(firsts)
                d2s = []
                gidxs = []
                for u in range(UNROLL):
                    c = g * UNROLL + u
                    xc = xv[pl.ds(c * 16, 16)]
                    yc = yv[pl.ds(c * 16, 16)]
                    zc = zv[pl.ds(c * 16, 16)]
                    dx = xc - cx
                    dy = yc - cy
                    dz = zc - cz
                    d2s.append(dx * dx + dy * dy + dz * dz)
                    gidxs.append(lane + c * 16)
                for i in range(3):
                    for u in range(UNROLL):
                        m = d2s[u] <= R2[i]
                        cs = jnp.cumsum(m.astype(jnp.int32))
                        slot = cnts[i] + cs - 1
                        valid = jnp.logical_and(m, slot < KS[i])
                        slot = jnp.minimum(jnp.maximum(slot, 0), KS[i] - 1)
                        plsc.store_scatter(ibufs[i], [slot], gidxs[u],
                                           mask=valid)
                        cnts[i] = cnts[i] + plsc.all_reduce_population_count(m)
                        firsts[i] = jnp.minimum(
                            firsts[i], jnp.where(m, gidxs[u], N - 1))
                return tuple(cnts), tuple(firsts)

            z16 = jnp.zeros((16,), jnp.int32)
            f16 = jnp.full((16,), N - 1, jnp.int32)
            cnts, firsts = lax.fori_loop(
                0, N // (16 * UNROLL), scan_group,
                ((z16, z16, z16), (f16, f16, f16)))
            for i in range(3):
                first = jnp.full((16,), jnp.min(firsts[i]), jnp.int32)
                for j in range(KS[i] // 16):
                    cur = ibufs[i][pl.ds(16 * j, 16)]
                    res = jnp.where(lane + 16 * j < cnts[i], cur, first)
                    off = XOFF[i] + 16 * j
                    outb[pl.ds(off, 16)] = plsc.load_gather(xv, [res]) - cx
                    outb[pl.ds(KTOT + off, 16)] = plsc.load_gather(yv, [res]) - cy
                    outb[pl.ds(2 * KTOT + off, 16)] = plsc.load_gather(zv, [res]) - cz
            pltpu.sync_copy(outb, g_hbm.at[b * S + s_abs])
            return 0

        lax.fori_loop(0, 128, per_center, 0)

    return select


_select_cache = []


def _get_select():
    if not _select_cache:
        _select_cache.append(_make_select())
    return _select_cache[0]


SBLK = 64


def _mlp_body(g_ref, *refs):
    wrefs = refs[:18]
    o_refs = refs[18:]
    sb = pl.program_id(1)
    r = g_ref[0]
    for i in range(3):
        k = KS[i]
        o = XOFF[i]
        gx = r[:, o:o + k]
        gy = r[:, KTOT + o:KTOT + o + k]
        gz = r[:, 2 * KTOT + o:2 * KTOT + o + k]
        w1, b1, w2, b2, w3, b3 = wrefs[6 * i:6 * i + 6]
        h = (gx[:, :, None] * w1[0][None, None, :]
             + gy[:, :, None] * w1[1][None, None, :]
             + gz[:, :, None] * w1[2][None, None, :]
             + b1[0][None, None, :])
        h = jnp.maximum(h, 0.0)
        c1 = h.shape[-1]
        h = h.reshape(SBLK * k, c1)
        h = jnp.maximum(jnp.dot(h, w2[...], preferred_element_type=jnp.float32)
                        + b2[...], 0.0)
        h = jnp.maximum(jnp.dot(h, w3[...], preferred_element_type=jnp.float32)
                        + b3[...], 0.0)
        m = jnp.max(h.reshape(SBLK, k, 128), axis=0)
        o_ref = o_refs[i]

        @pl.when(sb == 0)
        def _():
            o_ref[0] = m

        @pl.when(sb != 0)
        def _():
            o_ref[0] = jnp.maximum(o_ref[0], m)


def _mlp(g, wlist):
    grid = (B, S // SBLK)
    in_specs = [pl.BlockSpec((1, SBLK, GW), lambda bb, sb: (bb, sb, 0))]
    for w in wlist:
        in_specs.append(
            pl.BlockSpec(w.shape, lambda bb, sb, nd=w.ndim: (0,) * nd))
    out_specs = [
        pl.BlockSpec((1, KS[i], 128), lambda bb, sb: (bb, 0, 0))
        for i in range(3)
    ]
    return pl.pallas_call(
        _mlp_body,
        grid=grid,
        in_specs=in_specs,
        out_specs=out_specs,
        out_shape=[
            jax.ShapeDtypeStruct((B, KS[i], 128), jnp.float32) for i in range(3)
        ],
    )(g, *wlist)


def kernel(xyz, points, params):
    del points
    new_xyz, fps_idx = _fps(xyz)
    g = _get_select()(xyz.reshape(B * 3, N), fps_idx)
    wlist = []
    for i in range(3):
        for (w, bvec) in params[i]:
            wlist.append(w)
            wlist.append(bvec.reshape(1, -1))
    o1, o2, o3 = _mlp(g.reshape(B, S, GW), wlist)
    return new_xyz, jnp.concatenate([o1, o2, o3], axis=1)


# speedup vs baseline: 21.9632x; 1.0156x over previous
import functools

import jax
import jax.numpy as jnp
import numpy as np
from jax import lax
from jax.experimental import pallas as pl
from jax.experimental.pallas import tpu as pltpu
from jax.experimental.pallas import tpu_sc as plsc

B = 8
N = 4096
S = 512
KS = (16, 32, 64)
R2 = tuple(np.float32(r * r) for r in (0.1, 0.2, 0.4))
KTOT = sum(KS)
GW = 3 * KTOT
XOFF = (0, KS[0], KS[0] + KS[1])


def _fps_body(xyz_ref, new_ref, idx_ref):
    x = xyz_ref[:, 0, :]
    y = xyz_ref[:, 1, :]
    z = xyz_ref[:, 2, :]
    iota = lax.broadcasted_iota(jnp.int32, (B, N), 1)
    slot = (lax.broadcasted_iota(jnp.int32, (B, S), 1)
            + lax.broadcasted_iota(jnp.int32, (B, S), 0) * S) & (S - 1)

    def step(t, carry):
        dist, idx, cx, cy, cz, ia, xa, ya, za = carry
        here = slot == t
        hi = here.astype(jnp.int32)
        hf = here.astype(jnp.float32)
        ia = ia + hi * idx
        xa = xa + hf * cx
        ya = ya + hf * cy
        za = za + hf * cz
        d = (x - cx) ** 2 + (y - cy) ** 2 + (z - cz) ** 2
        dist = jnp.minimum(dist, d)
        m = jnp.max(dist, axis=1, keepdims=True)
        cand = jnp.where(dist == m, iota, N)
        nidx = jnp.min(cand, axis=1, keepdims=True)
        sel = iota == nidx
        zf = jnp.zeros((), jnp.float32)
        ncx = jnp.sum(jnp.where(sel, x, zf), axis=1, keepdims=True)
        ncy = jnp.sum(jnp.where(sel, y, zf), axis=1, keepdims=True)
        ncz = jnp.sum(jnp.where(sel, z, zf), axis=1, keepdims=True)
        return dist, nidx, ncx, ncy, ncz, ia, xa, ya, za

    init = (
        jnp.full((B, N), 1e10, jnp.float32),
        jnp.zeros((B, 1), jnp.int32),
        x[:, 0:1],
        y[:, 0:1],
        z[:, 0:1],
        jnp.zeros((B, S), jnp.int32),
        jnp.zeros((B, S), jnp.float32),
        jnp.zeros((B, S), jnp.float32),
        jnp.zeros((B, S), jnp.float32),
    )
    res = lax.fori_loop(0, S, step, init)
    idx_ref[...] = res[5]
    new_ref[:, 0, :] = res[6]
    new_ref[:, 1, :] = res[7]
    new_ref[:, 2, :] = res[8]


def _fps(xyz):
    return pl.pallas_call(
        _fps_body,
        out_shape=[
            jax.ShapeDtypeStruct((B, 3, S), jnp.float32),
            jax.ShapeDtypeStruct((B, S), jnp.int32),
        ],
    )(xyz)


def _make_select():
    mesh = plsc.VectorSubcoreMesh(core_axis_name="c", subcore_axis_name="s")

    @functools.partial(
        pl.kernel,
        mesh=mesh,
        compiler_params=pltpu.CompilerParams(needs_layout_passes=False),
        out_type=jax.ShapeDtypeStruct((B * S, GW), jnp.float32),
        scratch_types=[
            pltpu.VMEM((N,), jnp.float32),
            pltpu.VMEM((N,), jnp.float32),
            pltpu.VMEM((N,), jnp.float32),
            pltpu.VMEM((S,), jnp.int32),
            pltpu.VMEM((GW,), jnp.float32),
            pltpu.VMEM((KS[0],), jnp.int32),
            pltpu.VMEM((KS[1],), jnp.int32),
            pltpu.VMEM((KS[2],), jnp.int32),
        ],
    )
    def select(xyz_hbm, fps_hbm, g_hbm, xv, yv, zv, fpsv, outb, ib1, ib2, ib3):
        wid = lax.axis_index("s") * 2 + lax.axis_index("c")
        b = wid // 4
        q = wid % 4
        pltpu.sync_copy(xyz_hbm.at[b * 3 + 0], xv)
        pltpu.sync_copy(xyz_hbm.at[b * 3 + 1], yv)
        pltpu.sync_copy(xyz_hbm.at[b * 3 + 2], zv)
        pltpu.sync_copy(fps_hbm.at[b], fpsv)
        lane = jnp.arange(16, dtype=jnp.int32)
        ibufs = (ib1, ib2, ib3)

        def per_center(s_loc, _):
            s_abs = q * 128 + s_loc
            cidx = plsc.load_gather(fpsv, [jnp.full((16,), s_abs, jnp.int32)])
            cx = plsc.load_gather(xv, [cidx])
            cy = plsc.load_gather(yv, [cidx])
            cz = plsc.load_gather(zv, [cidx])
            fill = jnp.full((16,), N - 1, jnp.int32)
            for i in range(3):
                for j in range(KS[i] // 16):
                    ibufs[i][pl.ds(16 * j, 16)] = fill

            UNROLL = 8

            def scan_group(g, carry):
                cnts, firsts = carry
                cnts = list(cnts)
                firsts = list(firsts)
                d2s = []
                gidxs = []
                for u in range(UNROLL):
                    c = g * UNROLL + u
                    xc = xv[pl.ds(c * 16, 16)]
                    yc = yv[pl.ds(c * 16, 16)]
                    zc = zv[pl.ds(c * 16, 16)]
                    dx = xc - cx
                    dy = yc - cy
                    dz = zc - cz
                    d2s.append(dx * dx + dy * dy + dz * dz)
                    gidxs.append(lane + c * 16)
                for i in range(3):
                    for u in range(UNROLL):
                        m = d2s[u] <= R2[i]
                        cs = jnp.cumsum(m.astype(jnp.int32))
                        slot = cnts[i] + cs - 1
                        valid = jnp.logical_and(m, slot < KS[i])
                        slot = jnp.minimum(jnp.maximum(slot, 0), KS[i] - 1)
                        plsc.store_scatter(ibufs[i], [slot], gidxs[u],
                                           mask=valid)
                        cnts[i] = cnts[i] + plsc.all_reduce_population_count(m)
                        firsts[i] = jnp.minimum(
                            firsts[i], jnp.where(m, gidxs[u], N - 1))
                return tuple(cnts), tuple(firsts)

            z16 = jnp.zeros((16,), jnp.int32)
            f16 = jnp.full((16,), N - 1, jnp.int32)
            cnts, firsts = lax.fori_loop(
                0, N // (16 * UNROLL), scan_group,
                ((z16, z16, z16), (f16, f16, f16)))
            for i in range(3):
                first = jnp.full((16,), jnp.min(firsts[i]), jnp.int32)
                for j in range(KS[i] // 16):
                    cur = ibufs[i][pl.ds(16 * j, 16)]
                    res = jnp.where(lane + 16 * j < cnts[i], cur, first)
                    off = XOFF[i] + 16 * j
                    outb[pl.ds(off, 16)] = plsc.load_gather(xv, [res]) - cx
                    outb[pl.ds(KTOT + off, 16)] = plsc.load_gather(yv, [res]) - cy
                    outb[pl.ds(2 * KTOT + off, 16)] = plsc.load_gather(zv, [res]) - cz
            pltpu.sync_copy(outb, g_hbm.at[b * S + s_abs])
            return 0

        lax.fori_loop(0, 128, per_center, 0)

    return select


_select_cache = []


def _get_select():
    if not _select_cache:
        _select_cache.append(_make_select())
    return _select_cache[0]


SBLK = 64


def _mlp_body(g_ref, *refs):
    wrefs = refs[:18]
    o_refs = refs[18:]
    sb = pl.program_id(1)
    r = g_ref[0]
    for i in range(3):
        k = KS[i]
        o = XOFF[i]
        gx = r[:, o:o + k]
        gy = r[:, KTOT + o:KTOT + o + k]
        gz = r[:, 2 * KTOT + o:2 * KTOT + o + k]
        w1, b1, w2, b2, w3, b3 = wrefs[6 * i:6 * i + 6]
        h = (gx[:, :, None] * w1[0][None, None, :]
             + gy[:, :, None] * w1[1][None, None, :]
             + gz[:, :, None] * w1[2][None, None, :]
             + b1[0][None, None, :])
        h = jnp.maximum(h, 0.0)
        c1 = h.shape[-1]
        h = h.reshape(SBLK * k, c1)
        h = jnp.maximum(jnp.dot(h, w2[...], preferred_element_type=jnp.float32)
                        + b2[...], 0.0)
        h = jnp.maximum(jnp.dot(h, w3[...], preferred_element_type=jnp.float32)
                        + b3[...], 0.0)
        m = jnp.max(h.reshape(SBLK, k, 128), axis=0)
        o_ref = o_refs[i]

        @pl.when(sb == 0)
        def _():
            o_ref[0] = m

        @pl.when(sb != 0)
        def _():
            o_ref[0] = jnp.maximum(o_ref[0], m)


def _mlp(g, wlist):
    grid = (B, S // SBLK)
    in_specs = [pl.BlockSpec((1, SBLK, GW), lambda bb, sb: (bb, sb, 0))]
    for w in wlist:
        in_specs.append(
            pl.BlockSpec(w.shape, lambda bb, sb, nd=w.ndim: (0,) * nd))
    out_specs = [
        pl.BlockSpec((1, KS[i], 128), lambda bb, sb: (bb, 0, 0))
        for i in range(3)
    ]
    return pl.pallas_call(
        _mlp_body,
        grid=grid,
        in_specs=in_specs,
        out_specs=out_specs,
        out_shape=[
            jax.ShapeDtypeStruct((B, KS[i], 128), jnp.float32) for i in range(3)
        ],
    )(g, *wlist)


def kernel(xyz, points, params):
    del points
    new_xyz, fps_idx = _fps(xyz)
    g = _get_select()(xyz.reshape(B * 3, N), fps_idx)
    wlist = []
    for i in range(3):
        for (w, bvec) in params[i]:
            wlist.append(w)
            wlist.append(bvec.reshape(1, -1))
    o1, o2, o3 = _mlp(g.reshape(B, S, GW), wlist)
    return new_xyz, jnp.concatenate([o1, o2, o3], axis=1)


# speedup vs baseline: 23.3831x; 1.0646x over previous
import functools

import jax
import jax.numpy as jnp
import numpy as np
from jax import lax
from jax.experimental import pallas as pl
from jax.experimental.pallas import tpu as pltpu
from jax.experimental.pallas import tpu_sc as plsc

B = 8
N = 4096
S = 512
KS = (16, 32, 64)
R2 = tuple(np.float32(r * r) for r in (0.1, 0.2, 0.4))
KTOT = sum(KS)
GW = 3 * KTOT
XOFF = (0, KS[0], KS[0] + KS[1])


def _fps_body(xyz_ref, new_ref, idx_ref):
    x = xyz_ref[:, 0, :]
    y = xyz_ref[:, 1, :]
    z = xyz_ref[:, 2, :]
    iota = lax.broadcasted_iota(jnp.int32, (B, N), 1)
    slot = (lax.broadcasted_iota(jnp.int32, (B, S), 1)
            + lax.broadcasted_iota(jnp.int32, (B, S), 0) * S) & (S - 1)

    def step(t, carry):
        dist, idx, cx, cy, cz, ia, xa, ya, za = carry
        here = slot == t
        hi = here.astype(jnp.int32)
        hf = here.astype(jnp.float32)
        ia = ia + hi * idx
        xa = xa + hf * cx
        ya = ya + hf * cy
        za = za + hf * cz
        d = (x - cx) ** 2 + (y - cy) ** 2 + (z - cz) ** 2
        dist = jnp.minimum(dist, d)
        nidx = jnp.argmax(dist, axis=1).astype(jnp.int32)[:, None]
        sel = iota == nidx
        zf = jnp.zeros((), jnp.float32)
        xyz3 = jnp.stack([x, y, z])
        csum = jnp.sum(jnp.where(sel[None], xyz3, zf), axis=2)
        ncx = csum[0][:, None]
        ncy = csum[1][:, None]
        ncz = csum[2][:, None]
        return dist, nidx, ncx, ncy, ncz, ia, xa, ya, za

    init = (
        jnp.full((B, N), 1e10, jnp.float32),
        jnp.zeros((B, 1), jnp.int32),
        x[:, 0:1],
        y[:, 0:1],
        z[:, 0:1],
        jnp.zeros((B, S), jnp.int32),
        jnp.zeros((B, S), jnp.float32),
        jnp.zeros((B, S), jnp.float32),
        jnp.zeros((B, S), jnp.float32),
    )
    res = lax.fori_loop(0, S, step, init)
    idx_ref[...] = res[5]
    new_ref[:, 0, :] = res[6]
    new_ref[:, 1, :] = res[7]
    new_ref[:, 2, :] = res[8]


def _fps(xyz):
    return pl.pallas_call(
        _fps_body,
        out_shape=[
            jax.ShapeDtypeStruct((B, 3, S), jnp.float32),
            jax.ShapeDtypeStruct((B, S), jnp.int32),
        ],
    )(xyz)


def _make_select():
    mesh = plsc.VectorSubcoreMesh(core_axis_name="c", subcore_axis_name="s")

    @functools.partial(
        pl.kernel,
        mesh=mesh,
        compiler_params=pltpu.CompilerParams(needs_layout_passes=False),
        out_type=jax.ShapeDtypeStruct((B * S, GW), jnp.float32),
        scratch_types=[
            pltpu.VMEM((N,), jnp.float32),
            pltpu.VMEM((N,), jnp.float32),
            pltpu.VMEM((N,), jnp.float32),
            pltpu.VMEM((S,), jnp.int32),
            pltpu.VMEM((GW,), jnp.float32),
            pltpu.VMEM((KS[0],), jnp.int32),
            pltpu.VMEM((KS[1],), jnp.int32),
            pltpu.VMEM((KS[2],), jnp.int32),
        ],
    )
    def select(xyz_hbm, fps_hbm, g_hbm, xv, yv, zv, fpsv, outb, ib1, ib2, ib3):
        wid = lax.axis_index("s") * 2 + lax.axis_index("c")
        b = wid // 4
        q = wid % 4
        pltpu.sync_copy(xyz_hbm.at[b * 3 + 0], xv)
        pltpu.sync_copy(xyz_hbm.at[b * 3 + 1], yv)
        pltpu.sync_copy(xyz_hbm.at[b * 3 + 2], zv)
        pltpu.sync_copy(fps_hbm.at[b], fpsv)
        lane = jnp.arange(16, dtype=jnp.int32)
        ibufs = (ib1, ib2, ib3)

        def per_center(s_loc, _):
            s_abs = q * 128 + s_loc
            cidx = plsc.load_gather(fpsv, [jnp.full((16,), s_abs, jnp.int32)])
            cx = plsc.load_gather(xv, [cidx])
            cy = plsc.load_gather(yv, [cidx])
            cz = plsc.load_gather(zv, [cidx])
            fill = jnp.full((16,), N - 1, jnp.int32)
            for i in range(3):
                for j in range(KS[i] // 16):
                    ibufs[i][pl.ds(16 * j, 16)] = fill

            UNROLL = 8

            def scan_group(g, carry):
                cnts, firsts = carry
                cnts = list(cnts)
                firsts = list(firsts)
                d2s = []
                gidxs = []
                for u in range(UNROLL):
                    c = g * UNROLL + u
                    xc = xv[pl.ds(c * 16, 16)]
                    yc = yv[pl.ds(c * 16, 16)]
                    zc = zv[pl.ds(c * 16, 16)]
                    dx = xc - cx
                    dy = yc - cy
                    dz = zc - cz
                    d2s.append(dx * dx + dy * dy + dz * dz)
                    gidxs.append(lane + c * 16)
                for i in range(3):
                    for u in range(UNROLL):
                        m = d2s[u] <= R2[i]
                        cs = jnp.cumsum(m.astype(jnp.int32))
                        slot = cnts[i] + cs - 1
                        valid = jnp.logical_and(m, slot < KS[i])
                        slot = jnp.minimum(jnp.maximum(slot, 0), KS[i] - 1)
                        plsc.store_scatter(ibufs[i], [slot], gidxs[u],
                                           mask=valid)
                        cnts[i] = cnts[i] + plsc.all_reduce_population_count(m)
                        firsts[i] = jnp.minimum(
                            firsts[i], jnp.where(m, gidxs[u], N - 1))
                return tuple(cnts), tuple(firsts)

            z16 = jnp.zeros((16,), jnp.int32)
            f16 = jnp.full((16,), N - 1, jnp.int32)
            cnts, firsts = lax.fori_loop(
                0, N // (16 * UNROLL), scan_group,
                ((z16, z16, z16), (f16, f16, f16)))
            for i in range(3):
                first = jnp.full((16,), jnp.min(firsts[i]), jnp.int32)
                for j in range(KS[i] // 16):
                    cur = ibufs[i][pl.ds(16 * j, 16)]
                    res = jnp.where(lane + 16 * j < cnts[i], cur, first)
                    off = XOFF[i] + 16 * j
                    outb[pl.ds(off, 16)] = plsc.load_gather(xv, [res]) - cx
                    outb[pl.ds(KTOT + off, 16)] = plsc.load_gather(yv, [res]) - cy
                    outb[pl.ds(2 * KTOT + off, 16)] = plsc.load_gather(zv, [res]) - cz
            pltpu.sync_copy(outb, g_hbm.at[b * S + s_abs])
            return 0

        lax.fori_loop(0, 128, per_center, 0)

    return select


_select_cache = []


def _get_select():
    if not _select_cache:
        _select_cache.append(_make_select())
    return _select_cache[0]


SBLK = 64


def _mlp_body(g_ref, *refs):
    wrefs = refs[:18]
    o_refs = refs[18:]
    sb = pl.program_id(1)
    r = g_ref[0]
    for i in range(3):
        k = KS[i]
        o = XOFF[i]
        gx = r[:, o:o + k]
        gy = r[:, KTOT + o:KTOT + o + k]
        gz = r[:, 2 * KTOT + o:2 * KTOT + o + k]
        w1, b1, w2, b2, w3, b3 = wrefs[6 * i:6 * i + 6]
        h = (gx[:, :, None] * w1[0][None, None, :]
             + gy[:, :, None] * w1[1][None, None, :]
             + gz[:, :, None] * w1[2][None, None, :]
             + b1[0][None, None, :])
        h = jnp.maximum(h, 0.0)
        c1 = h.shape[-1]
        h = h.reshape(SBLK * k, c1)
        h = jnp.maximum(jnp.dot(h, w2[...], preferred_element_type=jnp.float32)
                        + b2[...], 0.0)
        h = jnp.maximum(jnp.dot(h, w3[...], preferred_element_type=jnp.float32)
                        + b3[...], 0.0)
        m = jnp.max(h.reshape(SBLK, k, 128), axis=0)
        o_ref = o_refs[i]

        @pl.when(sb == 0)
        def _():
            o_ref[0] = m

        @pl.when(sb != 0)
        def _():
            o_ref[0] = jnp.maximum(o_ref[0], m)


def _mlp(g, wlist):
    grid = (B, S // SBLK)
    in_specs = [pl.BlockSpec((1, SBLK, GW), lambda bb, sb: (bb, sb, 0))]
    for w in wlist:
        in_specs.append(
            pl.BlockSpec(w.shape, lambda bb, sb, nd=w.ndim: (0,) * nd))
    out_specs = [
        pl.BlockSpec((1, KS[i], 128), lambda bb, sb: (bb, 0, 0))
        for i in range(3)
    ]
    return pl.pallas_call(
        _mlp_body,
        grid=grid,
        in_specs=in_specs,
        out_specs=out_specs,
        out_shape=[
            jax.ShapeDtypeStruct((B, KS[i], 128), jnp.float32) for i in range(3)
        ],
    )(g, *wlist)


def kernel(xyz, points, params):
    del points
    new_xyz, fps_idx = _fps(xyz)
    g = _get_select()(xyz.reshape(B * 3, N), fps_idx)
    wlist = []
    for i in range(3):
        for (w, bvec) in params[i]:
            wlist.append(w)
            wlist.append(bvec.reshape(1, -1))
    o1, o2, o3 = _mlp(g.reshape(B, S, GW), wlist)
    return new_xyz, jnp.concatenate([o1, o2, o3], axis=1)


# speedup vs baseline: 24.0154x; 1.0270x over previous
import functools

import jax
import jax.numpy as jnp
import numpy as np
from jax import lax
from jax.experimental import pallas as pl
from jax.experimental.pallas import tpu as pltpu
from jax.experimental.pallas import tpu_sc as plsc

B = 8
N = 4096
S = 512
KS = (16, 32, 64)
R2 = tuple(np.float32(r * r) for r in (0.1, 0.2, 0.4))
KTOT = sum(KS)
GW = 3 * KTOT
XOFF = (0, KS[0], KS[0] + KS[1])


def _fps_body(xyz_ref, new_ref, idx_ref):
    x = xyz_ref[:, 0, :]
    y = xyz_ref[:, 1, :]
    z = xyz_ref[:, 2, :]
    iota = lax.broadcasted_iota(jnp.int32, (B, N), 1)
    slot = (lax.broadcasted_iota(jnp.int32, (B, S), 1)
            + lax.broadcasted_iota(jnp.int32, (B, S), 0) * S) & (S - 1)

    def step(t, carry):
        dist, idx, cx, cy, cz, ia, xa, ya, za = carry
        here = slot == t
        hi = here.astype(jnp.int32)
        hf = here.astype(jnp.float32)
        ia = ia + hi * idx
        xa = xa + hf * cx
        ya = ya + hf * cy
        za = za + hf * cz
        d = (x - cx) ** 2 + (y - cy) ** 2 + (z - cz) ** 2
        dist = jnp.minimum(dist, d)
        nidx = jnp.argmax(dist, axis=1).astype(jnp.int32)[:, None]
        sel = iota == nidx
        zf = jnp.zeros((), jnp.float32)
        xyz3 = jnp.stack([x, y, z])
        csum = jnp.sum(jnp.where(sel[None], xyz3, zf), axis=2)
        ncx = csum[0][:, None]
        ncy = csum[1][:, None]
        ncz = csum[2][:, None]
        return dist, nidx, ncx, ncy, ncz, ia, xa, ya, za

    init = (
        jnp.full((B, N), 1e10, jnp.float32),
        jnp.zeros((B, 1), jnp.int32),
        x[:, 0:1],
        y[:, 0:1],
        z[:, 0:1],
        jnp.zeros((B, S), jnp.int32),
        jnp.zeros((B, S), jnp.float32),
        jnp.zeros((B, S), jnp.float32),
        jnp.zeros((B, S), jnp.float32),
    )
    res = lax.fori_loop(0, S, step, init)
    idx_ref[...] = res[5]
    new_ref[:, 0, :] = res[6]
    new_ref[:, 1, :] = res[7]
    new_ref[:, 2, :] = res[8]


def _fps(xyz):
    return pl.pallas_call(
        _fps_body,
        out_shape=[
            jax.ShapeDtypeStruct((B, 3, S), jnp.float32),
            jax.ShapeDtypeStruct((B, S), jnp.int32),
        ],
    )(xyz)


def _make_select():
    mesh = plsc.VectorSubcoreMesh(core_axis_name="c", subcore_axis_name="s")

    @functools.partial(
        pl.kernel,
        mesh=mesh,
        compiler_params=pltpu.CompilerParams(needs_layout_passes=False),
        out_type=jax.ShapeDtypeStruct((B * S, GW), jnp.float32),
        scratch_types=[
            pltpu.VMEM((N,), jnp.float32),
            pltpu.VMEM((N,), jnp.float32),
            pltpu.VMEM((N,), jnp.float32),
            pltpu.VMEM((S,), jnp.int32),
            pltpu.VMEM((GW,), jnp.float32),
            pltpu.VMEM((KS[0],), jnp.int32),
            pltpu.VMEM((KS[1],), jnp.int32),
            pltpu.VMEM((KS[2],), jnp.int32),
        ],
    )
    def select(xyz_hbm, fps_hbm, g_hbm, xv, yv, zv, fpsv, outb, ib1, ib2, ib3):
        wid = lax.axis_index("s") * 2 + lax.axis_index("c")
        b = wid // 4
        q = wid % 4
        pltpu.sync_copy(xyz_hbm.at[b * 3 + 0], xv)
        pltpu.sync_copy(xyz_hbm.at[b * 3 + 1], yv)
        pltpu.sync_copy(xyz_hbm.at[b * 3 + 2], zv)
        pltpu.sync_copy(fps_hbm.at[b], fpsv)
        lane = jnp.arange(16, dtype=jnp.int32)
        ibufs = (ib1, ib2, ib3)

        def per_center(s_loc, _):
            s_abs = q * 128 + s_loc
            cidx = plsc.load_gather(fpsv, [jnp.full((16,), s_abs, jnp.int32)])
            cx = plsc.load_gather(xv, [cidx])
            cy = plsc.load_gather(yv, [cidx])
            cz = plsc.load_gather(zv, [cidx])
            fill = jnp.full((16,), N - 1, jnp.int32)
            for i in range(3):
                for j in range(KS[i] // 16):
                    ibufs[i][pl.ds(16 * j, 16)] = fill

            UNROLL = 8

            def scan_group(g, carry):
                cnts, firsts = carry
                cnts = list(cnts)
                firsts = list(firsts)
                d2s = []
                gidxs = []
                for u in range(UNROLL):
                    c = g * UNROLL + u
                    xc = xv[pl.ds(c * 16, 16)]
                    yc = yv[pl.ds(c * 16, 16)]
                    zc = zv[pl.ds(c * 16, 16)]
                    dx = xc - cx
                    dy = yc - cy
                    dz = zc - cz
                    d2s.append(dx * dx + dy * dy + dz * dz)
                    gidxs.append(lane + c * 16)
                for i in range(3):
                    for u in range(UNROLL):
                        m = d2s[u] <= R2[i]
                        cs = jnp.cumsum(m.astype(jnp.int32))
                        slot = cnts[i] + cs - 1
                        valid = jnp.logical_and(m, slot < KS[i])
                        slot = jnp.minimum(jnp.maximum(slot, 0), KS[i] - 1)
                        plsc.store_scatter(ibufs[i], [slot], gidxs[u],
                                           mask=valid)
                        cnts[i] = cnts[i] + plsc.all_reduce_population_count(m)
                        firsts[i] = jnp.minimum(
                            firsts[i], jnp.where(m, gidxs[u], N - 1))
                return tuple(cnts), tuple(firsts)

            z16 = jnp.zeros((16,), jnp.int32)
            f16 = jnp.full((16,), N - 1, jnp.int32)
            cnts, firsts = lax.fori_loop(
                0, N // (16 * UNROLL), scan_group,
                ((z16, z16, z16), (f16, f16, f16)))
            for i in range(3):
                first = jnp.full((16,), jnp.min(firsts[i]), jnp.int32)
                for j in range(KS[i] // 16):
                    cur = ibufs[i][pl.ds(16 * j, 16)]
                    res = jnp.where(lane + 16 * j < cnts[i], cur, first)
                    off = XOFF[i] + 16 * j
                    outb[pl.ds(off, 16)] = plsc.load_gather(xv, [res]) - cx
                    outb[pl.ds(KTOT + off, 16)] = plsc.load_gather(yv, [res]) - cy
                    outb[pl.ds(2 * KTOT + off, 16)] = plsc.load_gather(zv, [res]) - cz
            pltpu.sync_copy(outb, g_hbm.at[b * S + s_abs])
            return 0

        lax.fori_loop(0, 128, per_center, 0)

    return select


_select_cache = []


def _get_select():
    if not _select_cache:
        _select_cache.append(_make_select())
    return _select_cache[0]


SBLK = 128


def _mlp_body(g_ref, *refs):
    wrefs = refs[:18]
    o_refs = refs[18:]
    sb = pl.program_id(1)
    r = g_ref[0]
    for i in range(3):
        k = KS[i]
        o = XOFF[i]
        gx = r[:, o:o + k]
        gy = r[:, KTOT + o:KTOT + o + k]
        gz = r[:, 2 * KTOT + o:2 * KTOT + o + k]
        w1, b1, w2, b2, w3, b3 = wrefs[6 * i:6 * i + 6]
        h = (gx[:, :, None] * w1[0][None, None, :]
             + gy[:, :, None] * w1[1][None, None, :]
             + gz[:, :, None] * w1[2][None, None, :]
             + b1[0][None, None, :])
        h = jnp.maximum(h, 0.0)
        c1 = h.shape[-1]
        h = h.reshape(SBLK * k, c1)
        h = jnp.maximum(jnp.dot(h, w2[...], preferred_element_type=jnp.float32)
                        + b2[...], 0.0)
        h = jnp.maximum(jnp.dot(h, w3[...], preferred_element_type=jnp.float32)
                        + b3[...], 0.0)
        m = jnp.max(h.reshape(SBLK, k, 128), axis=0)
        o_ref = o_refs[i]

        @pl.when(sb == 0)
        def _():
            o_ref[0] = m

        @pl.when(sb != 0)
        def _():
            o_ref[0] = jnp.maximum(o_ref[0], m)


def _mlp(g, wlist):
    grid = (B, S // SBLK)
    in_specs = [pl.BlockSpec((1, SBLK, GW), lambda bb, sb: (bb, sb, 0))]
    for w in wlist:
        in_specs.append(
            pl.BlockSpec(w.shape, lambda bb, sb, nd=w.ndim: (0,) * nd))
    out_specs = [
        pl.BlockSpec((1, KS[i], 128), lambda bb, sb: (bb, 0, 0))
        for i in range(3)
    ]
    return pl.pallas_call(
        _mlp_body,
        grid=grid,
        in_specs=in_specs,
        out_specs=out_specs,
        out_shape=[
            jax.ShapeDtypeStruct((B, KS[i], 128), jnp.float32) for i in range(3)
        ],
    )(g, *wlist)


def kernel(xyz, points, params):
    del points
    new_xyz, fps_idx = _fps(xyz)
    g = _get_select()(xyz.reshape(B * 3, N), fps_idx)
    wlist = []
    for i in range(3):
        for (w, bvec) in params[i]:
            wlist.append(w)
            wlist.append(bvec.reshape(1, -1))
    o1, o2, o3 = _mlp(g.reshape(B, S, GW), wlist)
    return new_xyz, jnp.concatenate([o1, o2, o3], axis=1)


# speedup vs baseline: 24.9418x; 1.0386x over previous
import functools

import jax
import jax.numpy as jnp
import numpy as np
from jax import lax
from jax.experimental import pallas as pl
from jax.experimental.pallas import tpu as pltpu
from jax.experimental.pallas import tpu_sc as plsc

B = 8
N = 4096
S = 512
KS = (16, 32, 64)
R2 = tuple(np.float32(r * r) for r in (0.1, 0.2, 0.4))
KTOT = sum(KS)
GW = 3 * KTOT
XOFF = (0, KS[0], KS[0] + KS[1])


def _fps_body(xyz_ref, new_ref, idx_ref):
    x = xyz_ref[:, 0, :]
    y = xyz_ref[:, 1, :]
    z = xyz_ref[:, 2, :]
    iota = lax.broadcasted_iota(jnp.int32, (B, N), 1)
    slot = (lax.broadcasted_iota(jnp.int32, (B, S), 1)
            + lax.broadcasted_iota(jnp.int32, (B, S), 0) * S) & (S - 1)

    def step(t, carry):
        dist, idx, cx, cy, cz, ia, xa, ya, za = carry
        here = slot == t
        hi = here.astype(jnp.int32)
        hf = here.astype(jnp.float32)
        ia = ia + hi * idx
        xa = xa + hf * cx
        ya = ya + hf * cy
        za = za + hf * cz
        d = (x - cx) ** 2 + (y - cy) ** 2 + (z - cz) ** 2
        dist = jnp.minimum(dist, d)
        nidx = jnp.argmax(dist, axis=1).astype(jnp.int32)[:, None]
        sel = iota == nidx
        zf = jnp.zeros((), jnp.float32)
        xyz3 = jnp.stack([x, y, z])
        csum = jnp.sum(jnp.where(sel[None], xyz3, zf), axis=2)
        ncx = csum[0][:, None]
        ncy = csum[1][:, None]
        ncz = csum[2][:, None]
        return dist, nidx, ncx, ncy, ncz, ia, xa, ya, za

    init = (
        jnp.full((B, N), 1e10, jnp.float32),
        jnp.zeros((B, 1), jnp.int32),
        x[:, 0:1],
        y[:, 0:1],
        z[:, 0:1],
        jnp.zeros((B, S), jnp.int32),
        jnp.zeros((B, S), jnp.float32),
        jnp.zeros((B, S), jnp.float32),
        jnp.zeros((B, S), jnp.float32),
    )
    res = lax.fori_loop(0, S, step, init)
    idx_ref[...] = res[5]
    new_ref[:, 0, :] = res[6]
    new_ref[:, 1, :] = res[7]
    new_ref[:, 2, :] = res[8]


def _fps(xyz):
    return pl.pallas_call(
        _fps_body,
        out_shape=[
            jax.ShapeDtypeStruct((B, 3, S), jnp.float32),
            jax.ShapeDtypeStruct((B, S), jnp.int32),
        ],
    )(xyz)


def _make_select():
    mesh = plsc.VectorSubcoreMesh(core_axis_name="c", subcore_axis_name="s")

    @functools.partial(
        pl.kernel,
        mesh=mesh,
        compiler_params=pltpu.CompilerParams(needs_layout_passes=False),
        out_type=jax.ShapeDtypeStruct((B * S, GW), jnp.float32),
        scratch_types=[
            pltpu.VMEM((N,), jnp.float32),
            pltpu.VMEM((N,), jnp.float32),
            pltpu.VMEM((N,), jnp.float32),
            pltpu.VMEM((S,), jnp.int32),
            pltpu.VMEM((GW,), jnp.float32),
            pltpu.VMEM((KS[0],), jnp.int32),
            pltpu.VMEM((KS[1],), jnp.int32),
            pltpu.VMEM((KS[2],), jnp.int32),
        ],
    )
    def select(xyz_hbm, fps_hbm, g_hbm, xv, yv, zv, fpsv, outb, ib1, ib2, ib3):
        wid = lax.axis_index("s") * 2 + lax.axis_index("c")
        b = wid // 4
        q = wid % 4
        pltpu.sync_copy(xyz_hbm.at[b * 3 + 0], xv)
        pltpu.sync_copy(xyz_hbm.at[b * 3 + 1], yv)
        pltpu.sync_copy(xyz_hbm.at[b * 3 + 2], zv)
        pltpu.sync_copy(fps_hbm.at[b], fpsv)
        lane = jnp.arange(16, dtype=jnp.int32)
        ibufs = (ib1, ib2, ib3)

        def per_center(s_loc, _):
            s_abs = q * 128 + s_loc
            cidx = plsc.load_gather(fpsv, [jnp.full((16,), s_abs, jnp.int32)])
            cx = plsc.load_gather(xv, [cidx])
            cy = plsc.load_gather(yv, [cidx])
            cz = plsc.load_gather(zv, [cidx])
            fill = jnp.full((16,), N - 1, jnp.int32)
            for i in range(3):
                for j in range(KS[i] // 16):
                    ibufs[i][pl.ds(16 * j, 16)] = fill

            UNROLL = 8

            def scan_group(g, carry):
                cnts, firsts = carry
                cnts = list(cnts)
                firsts = list(firsts)
                d2s = []
                gidxs = []
                for u in range(UNROLL):
                    c = g * UNROLL + u
                    xc = xv[pl.ds(c * 16, 16)]
                    yc = yv[pl.ds(c * 16, 16)]
                    zc = zv[pl.ds(c * 16, 16)]
                    dx = xc - cx
                    dy = yc - cy
                    dz = zc - cz
                    d2s.append(dx * dx + dy * dy + dz * dz)
                    gidxs.append(lane + c * 16)
                for i in range(3):
                    def do_scale(i=i):
                        cnt = cnts[i]
                        fst = firsts[i]
                        for u in range(UNROLL):
                            m = d2s[u] <= R2[i]
                            cs = jnp.cumsum(m.astype(jnp.int32))
                            slot = cnt + cs - 1
                            valid = jnp.logical_and(m, slot < KS[i])
                            slot = jnp.minimum(jnp.maximum(slot, 0),
                                               KS[i] - 1)
                            plsc.store_scatter(ibufs[i], [slot], gidxs[u],
                                               mask=valid)
                            cnt = cnt + plsc.all_reduce_population_count(m)
                            fst = jnp.minimum(
                                fst, jnp.where(m, gidxs[u], N - 1))
                        return cnt, fst

                    def skip_scale(i=i):
                        return cnts[i], firsts[i]

                    pending = jnp.max(cnts[i]) < KS[i]
                    cnts[i], firsts[i] = lax.cond(pending, do_scale,
                                                  skip_scale)
                return tuple(cnts), tuple(firsts)

            z16 = jnp.zeros((16,), jnp.int32)
            f16 = jnp.full((16,), N - 1, jnp.int32)
            cnts, firsts = lax.fori_loop(
                0, N // (16 * UNROLL), scan_group,
                ((z16, z16, z16), (f16, f16, f16)))
            for i in range(3):
                first = jnp.full((16,), jnp.min(firsts[i]), jnp.int32)
                for j in range(KS[i] // 16):
                    cur = ibufs[i][pl.ds(16 * j, 16)]
                    res = jnp.where(lane + 16 * j < cnts[i], cur, first)
                    off = XOFF[i] + 16 * j
                    outb[pl.ds(off, 16)] = plsc.load_gather(xv, [res]) - cx
                    outb[pl.ds(KTOT + off, 16)] = plsc.load_gather(yv, [res]) - cy
                    outb[pl.ds(2 * KTOT + off, 16)] = plsc.load_gather(zv, [res]) - cz
            pltpu.sync_copy(outb, g_hbm.at[b * S + s_abs])
            return 0

        lax.fori_loop(0, 128, per_center, 0)

    return select


_select_cache = []


def _get_select():
    if not _select_cache:
        _select_cache.append(_make_select())
    return _select_cache[0]


SBLK = 128


def _mlp_body(g_ref, *refs):
    wrefs = refs[:18]
    o_refs = refs[18:]
    sb = pl.program_id(1)
    r = g_ref[0]
    for i in range(3):
        k = KS[i]
        o = XOFF[i]
        gx = r[:, o:o + k]
        gy = r[:, KTOT + o:KTOT + o + k]
        gz = r[:, 2 * KTOT + o:2 * KTOT + o + k]
        w1, b1, w2, b2, w3, b3 = wrefs[6 * i:6 * i + 6]
        h = (gx[:, :, None] * w1[0][None, None, :]
             + gy[:, :, None] * w1[1][None, None, :]
             + gz[:, :, None] * w1[2][None, None, :]
             + b1[0][None, None, :])
        h = jnp.maximum(h, 0.0)
        c1 = h.shape[-1]
        h = h.reshape(SBLK * k, c1)
        h = jnp.maximum(jnp.dot(h, w2[...], preferred_element_type=jnp.float32)
                        + b2[...], 0.0)
        h = jnp.maximum(jnp.dot(h, w3[...], preferred_element_type=jnp.float32)
                        + b3[...], 0.0)
        m = jnp.max(h.reshape(SBLK, k, 128), axis=0)
        o_ref = o_refs[i]

        @pl.when(sb == 0)
        def _():
            o_ref[0] = m

        @pl.when(sb != 0)
        def _():
            o_ref[0] = jnp.maximum(o_ref[0], m)


def _mlp(g, wlist):
    grid = (B, S // SBLK)
    in_specs = [pl.BlockSpec((1, SBLK, GW), lambda bb, sb: (bb, sb, 0))]
    for w in wlist:
        in_specs.append(
            pl.BlockSpec(w.shape, lambda bb, sb, nd=w.ndim: (0,) * nd))
    out_specs = [
        pl.BlockSpec((1, KS[i], 128), lambda bb, sb: (bb, 0, 0))
        for i in range(3)
    ]
    return pl.pallas_call(
        _mlp_body,
        grid=grid,
        in_specs=in_specs,
        out_specs=out_specs,
        out_shape=[
            jax.ShapeDtypeStruct((B, KS[i], 128), jnp.float32) for i in range(3)
        ],
    )(g, *wlist)


def kernel(xyz, points, params):
    del points
    new_xyz, fps_idx = _fps(xyz)
    g = _get_select()(xyz.reshape(B * 3, N), fps_idx)
    wlist = []
    for i in range(3):
        for (w, bvec) in params[i]:
            wlist.append(w)
            wlist.append(bvec.reshape(1, -1))
    o1, o2, o3 = _mlp(g.reshape(B, S, GW), wlist)
    return new_xyz, jnp.concatenate([o1, o2, o3], axis=1)
